# baseline (speedup 1.0000x reference)
import functools

import jax
import jax.numpy as jnp
from jax.experimental import pallas as pl
from jax.experimental.pallas import tpu as pltpu


def _pick_block(n, candidates):
    for c in candidates:
        if n % c == 0 and n // c >= 3:
            return c
    return n


def _layer0_body(x16_ref, adj_ref, w_ref, b_ref, adjl_ref, adjr_ref, h1_ref):
    ncol = adjl_ref.shape[1]
    a16 = adj_ref[...].astype(jnp.bfloat16)
    adjl_ref[...] = a16[:, :ncol]
    adjr_ref[...] = a16[:, ncol:]
    t = jnp.dot(a16, x16_ref[...], preferred_element_type=jnp.float32)
    out = (
        jnp.dot(t, w_ref[...], preferred_element_type=jnp.float32)
        + b_ref[...]
    )
    h1_ref[...] = out.astype(jnp.bfloat16)


def _rest_body(adjl_ref, adjr_ref, h1_ref, ws_ref, bs_ref, o_ref, h_ref, sem):
    l = pl.program_id(0)
    r = pl.program_id(1)
    nl = pl.num_programs(0)
    p = l % 2
    br = o_ref.shape[0]
    ncol = adjl_ref.shape[1]

    @pl.when(jnp.logical_and(l == 0, r == 0))
    def _():
        pltpu.make_async_copy(h1_ref, h_ref.at[0], sem).start()
        pltpu.make_async_copy(h1_ref, h_ref.at[0], sem).wait()

    hp = h_ref[p]
    t = jnp.dot(
        adjl_ref[pl.ds(r * br, br), :],
        hp[:ncol],
        preferred_element_type=jnp.float32,
    ) + jnp.dot(adjr_ref[...], hp[ncol:], preferred_element_type=jnp.float32)
    out = (
        jnp.dot(t, ws_ref[l], preferred_element_type=jnp.float32)
        + bs_ref[l][None, :]
    )
    h_ref[1 - p, pl.ds(r * br, br), :] = out.astype(jnp.bfloat16)

    @pl.when(l == nl - 1)
    def _():
        o_ref[...] = out.astype(jnp.bfloat16)


@functools.partial(jax.jit, static_argnames=("br0", "br", "ncol"))
def _gcn(x, adj, Ws, bs, br0, br, ncol):
    n, d = x.shape
    nl = Ws.shape[0]
    x16 = x.astype(jnp.bfloat16)

    adjl, adjr, h1 = pl.pallas_call(
        _layer0_body,
        grid=(n // br0,),
        in_specs=[
            pl.BlockSpec((n, d), lambda r: (0, 0)),
            pl.BlockSpec((br0, n), lambda r: (r, 0)),
            pl.BlockSpec((d, d), lambda r: (0, 0)),
            pl.BlockSpec((1, d), lambda r: (0, 0)),
        ],
        out_specs=[
            pl.BlockSpec((br0, ncol), lambda r: (r, 0)),
            pl.BlockSpec((br0, n - ncol), lambda r: (r, 0)),
            pl.BlockSpec((br0, d), lambda r: (r, 0)),
        ],
        out_shape=[
            jax.ShapeDtypeStruct((n, ncol), jnp.bfloat16),
            jax.ShapeDtypeStruct((n, n - ncol), jnp.bfloat16),
            jax.ShapeDtypeStruct((n, d), jnp.bfloat16),
        ],
        compiler_params=pltpu.CompilerParams(
            dimension_semantics=("parallel",),
        ),
    )(x16, adj, Ws[0], bs[0].reshape(1, d))

    out16 = pl.pallas_call(
        _rest_body,
        grid=(nl - 1, n // br),
        in_specs=[
            pl.BlockSpec((n, ncol), lambda l, r: (0, 0)),
            pl.BlockSpec((br, n - ncol), lambda l, r: (r, 0)),
            pl.BlockSpec(memory_space=pl.ANY),
            pl.BlockSpec((nl - 1, d, d), lambda l, r: (0, 0, 0)),
            pl.BlockSpec((nl - 1, d), lambda l, r: (0, 0)),
        ],
        out_specs=pl.BlockSpec(
            (br, d), lambda l, r: (jnp.where(l == nl - 2, r, 0), 0)
        ),
        out_shape=jax.ShapeDtypeStruct((n, d), jnp.bfloat16),
        scratch_shapes=[
            pltpu.VMEM((2, n, d), jnp.bfloat16),
            pltpu.SemaphoreType.DMA,
        ],
        compiler_params=pltpu.CompilerParams(
            dimension_semantics=("arbitrary", "arbitrary"),
            vmem_limit_bytes=110 * 1024 * 1024,
        ),
    )(adjl, adjr, h1, Ws[1:], bs[1:])
    return out16.astype(jnp.float32)


def kernel(x, adj, Ws, bs):
    n, _ = x.shape
    br0 = _pick_block(n, (400, 200, 80, 40, 16, 8))
    br = _pick_block(n, (1000, 400, 200, 80, 40, 16, 8))
    ncol = 1152 if n > 2000 else max(n // 4 // 8 * 8, 8)
    return _gcn(x, adj, Ws, bs, br0, br, ncol)
